# Initial kernel scaffold; baseline (speedup 1.0000x reference)
import jax
import jax.numpy as jnp
from jax.experimental import pallas as pl


def kernel(xyz, feats, W1, b1, g1, be1, W2, b2):
    raise NotImplementedError("write your pallas kernel here")


# speedup vs baseline: 11.4205x; 11.4205x over previous
import functools

import jax
import jax.numpy as jnp
from jax.experimental import pallas as pl
from jax.experimental.pallas import tpu as pltpu

B = 4
N = 4096
S = 1024
K = 32
R2 = 0.2 * 0.2
IN_CH = 64
OUT_CH = 128
HIDDEN = 64
T = 128
CTR_CHUNK = 8
INF = 1e30


def _fps_body(x_ref, y_ref, idx_ref, cx_ref, cy_ref):
    x = x_ref[...]
    y = y_ref[...]
    iota = jax.lax.broadcasted_iota(jnp.int32, (B, N), 1)
    iota_s = jax.lax.broadcasted_iota(jnp.int32, (B, S), 1)

    def body(i, carry):
        dists, far = carry
        oh = iota == far
        cx = jnp.sum(jnp.where(oh, x, 0.0), axis=1, keepdims=True)
        cy = jnp.sum(jnp.where(oh, y, 0.0), axis=1, keepdims=True)
        sel = iota_s == i
        idx_ref[...] = jnp.where(sel, far, idx_ref[...])
        cx_ref[...] = jnp.where(sel, cx, cx_ref[...])
        cy_ref[...] = jnp.where(sel, cy, cy_ref[...])
        d = (x - cx) ** 2 + (y - cy) ** 2
        dists = jnp.minimum(dists, d)
        m = jnp.max(dists, axis=1, keepdims=True)
        far = jnp.min(jnp.where(dists == m, iota, N), axis=1, keepdims=True)
        return dists, far

    init = (
        jnp.full((B, N), 1e10, jnp.float32),
        jnp.zeros((B, 1), jnp.int32),
    )
    jax.lax.fori_loop(0, S, body, init)


def _fps(x, y):
    return pl.pallas_call(
        _fps_body,
        out_shape=(
            jax.ShapeDtypeStruct((B, S), jnp.int32),
            jax.ShapeDtypeStruct((B, S), jnp.float32),
            jax.ShapeDtypeStruct((B, S), jnp.float32),
        ),
    )(x, y)


def _bq_body(x_ref, y_ref, cx_ref, cy_ref, idx_ref, val_ref, work_ref):
    x = x_ref[0]
    y = y_ref[0]
    cx = cx_ref[0, 0]
    cy = cy_ref[0, 0]
    d2 = (cx - x) ** 2 + (cy - y) ** 2
    within = d2 <= R2
    cnt = jnp.sum(within.astype(jnp.int32), axis=1, keepdims=True)
    masked = jnp.where(within, d2, INF)
    work_ref[...] = jnp.where(cnt == 0, d2, masked)
    iota = jax.lax.broadcasted_iota(jnp.int32, (T, N), 1)
    iota_k = jax.lax.broadcasted_iota(jnp.int32, (T, K), 1)

    def body(k, _):
        w = work_ref[...]
        m = jnp.min(w, axis=1, keepdims=True)
        am = jnp.min(jnp.where(w == m, iota, N), axis=1, keepdims=True)
        selk = iota_k == k
        idx_ref[0, 0] = jnp.where(selk, am, idx_ref[0, 0])
        val_ref[0, 0] = jnp.where(selk, (m < INF).astype(jnp.int32),
                                  val_ref[0, 0])
        work_ref[...] = jnp.where(iota == am, INF, w)
        return 0

    jax.lax.fori_loop(0, K, body, 0)


def _ball_query(x3, y3, cxc, cyc):
    grid = (B, S // T)
    return pl.pallas_call(
        _bq_body,
        grid=grid,
        in_specs=[
            pl.BlockSpec((1, 1, N), lambda b, t: (b, 0, 0)),
            pl.BlockSpec((1, 1, N), lambda b, t: (b, 0, 0)),
            pl.BlockSpec((1, 1, T, 1), lambda b, t: (b, t, 0, 0)),
            pl.BlockSpec((1, 1, T, 1), lambda b, t: (b, t, 0, 0)),
        ],
        out_specs=[
            pl.BlockSpec((1, 1, T, K), lambda b, t: (b, t, 0, 0)),
            pl.BlockSpec((1, 1, T, K), lambda b, t: (b, t, 0, 0)),
        ],
        out_shape=(
            jax.ShapeDtypeStruct((B, S // T, T, K), jnp.int32),
            jax.ShapeDtypeStruct((B, S // T, T, K), jnp.int32),
        ),
        scratch_shapes=[pltpu.VMEM((T, N), jnp.float32)],
    )(x3, y3, cxc, cyc)


def _pre_body(f_ref, xc_ref, yc_ref, w1f_ref, w1x_ref, w1y_ref, b1_ref, a_ref):
    f = f_ref[0]
    xc = xc_ref[0]
    yc = yc_ref[0]
    a = jnp.dot(f, w1f_ref[...], preferred_element_type=jnp.float32)
    a = a + xc * w1x_ref[...] + yc * w1y_ref[...] + b1_ref[...]
    a_ref[0] = a


def _precompute(feats, xcol, ycol, w1f, w1x, w1y, b1r):
    return pl.pallas_call(
        _pre_body,
        grid=(B,),
        in_specs=[
            pl.BlockSpec((1, N, IN_CH), lambda b: (b, 0, 0)),
            pl.BlockSpec((1, N, 1), lambda b: (b, 0, 0)),
            pl.BlockSpec((1, N, 1), lambda b: (b, 0, 0)),
            pl.BlockSpec((IN_CH, HIDDEN), lambda b: (0, 0)),
            pl.BlockSpec((1, HIDDEN), lambda b: (0, 0)),
            pl.BlockSpec((1, HIDDEN), lambda b: (0, 0)),
            pl.BlockSpec((1, HIDDEN), lambda b: (0, 0)),
        ],
        out_specs=pl.BlockSpec((1, N, HIDDEN), lambda b: (b, 0, 0)),
        out_shape=jax.ShapeDtypeStruct((B, N, HIDDEN), jnp.float32),
    )(feats, xcol, ycol, w1f, w1x, w1y, b1r)


def _mlp_body(a_ref, idx_ref, val_ref, cx_ref, cy_ref, w1x_ref, w1y_ref,
              g1_ref, be1_ref, w2_ref, b2_ref, out_ref):
    a = a_ref[0]
    idx = idx_ref[0, 0]
    val = val_ref[0, 0]
    cx = cx_ref[0, 0]
    cy = cy_ref[0, 0]
    cs = cx * w1x_ref[...] + cy * w1y_ref[...]
    g1 = g1_ref[...]
    be1 = be1_ref[...]
    w2 = w2_ref[...]
    b2 = b2_ref[...]

    iota_n = jax.lax.broadcasted_iota(jnp.int32, (CTR_CHUNK, K, N), 2)
    rows = CTR_CHUNK * K
    for c in range(T // CTR_CHUNK):
        sl = slice(c * CTR_CHUNK, (c + 1) * CTR_CHUNK)
        oh = (idx[sl][:, :, None] == iota_n).astype(jnp.float32)
        ag = jnp.dot(oh.reshape(rows, N), a,
                     preferred_element_type=jnp.float32)
        cs_rep = jnp.broadcast_to(cs[sl][:, None, :],
                                  (CTR_CHUNK, K, HIDDEN)).reshape(rows, HIDDEN)
        h = ag - cs_rep
        mu = jnp.mean(h, axis=1, keepdims=True)
        hm = h - mu
        var = jnp.mean(hm * hm, axis=1, keepdims=True)
        h = hm / jnp.sqrt(var + 1e-5) * g1 + be1
        h = jnp.maximum(h, 0.0)
        o = jnp.dot(h, w2, preferred_element_type=jnp.float32) + b2
        vmask = jnp.broadcast_to(val[sl][:, :, None] != 0,
                                 (CTR_CHUNK, K, OUT_CH))
        o = jnp.where(vmask, o.reshape(CTR_CHUNK, K, OUT_CH), -10000.0)
        out_ref[0, 0, sl, :] = jnp.max(o, axis=1)


def _mlp(a, idx, val, cxc, cyc, w1x, w1y, g1r, be1r, W2, b2r):
    grid = (B, S // T)
    return pl.pallas_call(
        _mlp_body,
        grid=grid,
        in_specs=[
            pl.BlockSpec((1, N, HIDDEN), lambda b, t: (b, 0, 0)),
            pl.BlockSpec((1, 1, T, K), lambda b, t: (b, t, 0, 0)),
            pl.BlockSpec((1, 1, T, K), lambda b, t: (b, t, 0, 0)),
            pl.BlockSpec((1, 1, T, 1), lambda b, t: (b, t, 0, 0)),
            pl.BlockSpec((1, 1, T, 1), lambda b, t: (b, t, 0, 0)),
            pl.BlockSpec((1, HIDDEN), lambda b, t: (0, 0)),
            pl.BlockSpec((1, HIDDEN), lambda b, t: (0, 0)),
            pl.BlockSpec((1, HIDDEN), lambda b, t: (0, 0)),
            pl.BlockSpec((1, HIDDEN), lambda b, t: (0, 0)),
            pl.BlockSpec((HIDDEN, OUT_CH), lambda b, t: (0, 0)),
            pl.BlockSpec((1, OUT_CH), lambda b, t: (0, 0)),
        ],
        out_specs=pl.BlockSpec((1, 1, T, OUT_CH), lambda b, t: (b, t, 0, 0)),
        out_shape=jax.ShapeDtypeStruct((B, S // T, T, OUT_CH), jnp.float32),
    )(a, idx, val, cxc, cyc, w1x, w1y, g1r, be1r, W2, b2r)


@jax.jit
def kernel(xyz, feats, W1, b1, g1, be1, W2, b2):
    x = xyz[:, :, 0]
    y = xyz[:, :, 1]
    x3 = x.reshape(B, 1, N)
    y3 = y.reshape(B, 1, N)
    xcol = x.reshape(B, N, 1)
    ycol = y.reshape(B, N, 1)
    w1x = W1[0].reshape(1, HIDDEN)
    w1y = W1[1].reshape(1, HIDDEN)
    w1f = W1[2:]
    b1r = b1.reshape(1, HIDDEN)
    g1r = g1.reshape(1, HIDDEN)
    be1r = be1.reshape(1, HIDDEN)
    b2r = b2.reshape(1, OUT_CH)

    fps_idx, cx, cy = _fps(x, y)
    cxc = cx.reshape(B, S // T, T, 1)
    cyc = cy.reshape(B, S // T, T, 1)
    idx, valmask = _ball_query(x3, y3, cxc, cyc)
    a = _precompute(feats, xcol, ycol, w1f, w1x, w1y, b1r)
    out = _mlp(a, idx, valmask, cxc, cyc, w1x, w1y, g1r, be1r, W2, b2r)

    centers = jnp.stack([cx, cy], axis=-1)
    return centers, out.reshape(B, S, OUT_CH)


# speedup vs baseline: 16.7184x; 1.4639x over previous
import functools

import jax
import jax.numpy as jnp
from jax import lax
from jax.experimental import pallas as pl
from jax.experimental.pallas import tpu as pltpu
from jax.experimental.pallas import tpu_sc as plsc

B = 4
N = 4096
S = 1024
K = 32
R2 = 0.2 * 0.2
IN_CH = 64
OUT_CH = 128
HIDDEN = 64
T = 128
INF = 1e30

TOT = B * S * K
NW = 32
ROWS_W = TOT // NW
CHUNK = 128
TW = 128
NCHUNK = ROWS_W // CHUNK


def _fps_body(xy_ref, cx_ref, cy_ref):
    xy = xy_ref[...]
    iota = jax.lax.broadcasted_iota(jnp.int32, (2 * B, N), 1)
    iota_s = jax.lax.broadcasted_iota(jnp.int32, (B, S), 1)
    fiota = jax.lax.broadcasted_iota(jnp.int32, (B, N), 1).astype(jnp.float32)

    def body(i, carry):
        dists, far = carry
        far2 = jnp.concatenate([far, far], axis=0).astype(jnp.int32)
        c = jnp.sum(jnp.where(iota == far2, xy, 0.0), axis=1, keepdims=True)
        sel = iota_s == i
        cx_ref[...] = jnp.where(sel, c[:B], cx_ref[...])
        cy_ref[...] = jnp.where(sel, c[B:], cy_ref[...])
        d2 = (xy - c) ** 2
        d = d2[:B] + d2[B:]
        dists = jnp.minimum(dists, d)
        m = jnp.max(dists, axis=1, keepdims=True)
        far = jnp.min(jnp.where(dists == m, fiota, float(N)),
                      axis=1, keepdims=True)
        return dists, far

    init = (
        jnp.full((B, N), 1e10, jnp.float32),
        jnp.zeros((B, 1), jnp.float32),
    )
    jax.lax.fori_loop(0, S, body, init)


def _fps(xy):
    return pl.pallas_call(
        _fps_body,
        out_shape=(
            jax.ShapeDtypeStruct((B, S), jnp.float32),
            jax.ShapeDtypeStruct((B, S), jnp.float32),
        ),
    )(xy)


def _bq_body(x_ref, y_ref, cx_ref, cy_ref, idx_ref, val_ref, work_ref):
    x = x_ref[0]
    y = y_ref[0]
    cx = cx_ref[0, 0]
    cy = cy_ref[0, 0]
    b = pl.program_id(0)
    d2 = (cx - x) ** 2 + (cy - y) ** 2
    within = d2 <= R2
    cnt = jnp.sum(jnp.where(within, 1.0, 0.0), axis=1, keepdims=True)
    masked = jnp.where(within, d2, INF)
    w0 = jnp.where(cnt == 0.0, d2, masked)
    work_ref[...] = w0
    m0 = jnp.min(w0, axis=1, keepdims=True)
    iota = jax.lax.broadcasted_iota(jnp.int32, (T, N), 1)
    fiota = iota.astype(jnp.float32)
    iota_k = jax.lax.broadcasted_iota(jnp.int32, (T, K), 1)

    def body(k, m):
        w = work_ref[...]
        amf = jnp.min(jnp.where(w == m, fiota, float(N)),
                      axis=1, keepdims=True)
        am = amf.astype(jnp.int32)
        selk = iota_k == k
        idx_ref[0, 0] = jnp.where(selk, am + b * N, idx_ref[0, 0])
        val_ref[0, 0] = jnp.where(selk, (m < INF).astype(jnp.int32),
                                  val_ref[0, 0])
        w = jnp.where(iota == am, INF, w)
        work_ref[...] = w
        return jnp.min(w, axis=1, keepdims=True)

    jax.lax.fori_loop(0, K, body, m0)


def _ball_query(x3, y3, cxc, cyc):
    grid = (B, S // T)
    return pl.pallas_call(
        _bq_body,
        grid=grid,
        in_specs=[
            pl.BlockSpec((1, 1, N), lambda b, t: (b, 0, 0)),
            pl.BlockSpec((1, 1, N), lambda b, t: (b, 0, 0)),
            pl.BlockSpec((1, 1, T, 1), lambda b, t: (b, t, 0, 0)),
            pl.BlockSpec((1, 1, T, 1), lambda b, t: (b, t, 0, 0)),
        ],
        out_specs=[
            pl.BlockSpec((1, 1, T, K), lambda b, t: (b, t, 0, 0)),
            pl.BlockSpec((1, 1, T, K), lambda b, t: (b, t, 0, 0)),
        ],
        out_shape=(
            jax.ShapeDtypeStruct((B, S // T, T, K), jnp.int32),
            jax.ShapeDtypeStruct((B, S // T, T, K), jnp.int32),
        ),
        scratch_shapes=[pltpu.VMEM((T, N), jnp.float32)],
    )(x3, y3, cxc, cyc)


def _pre_body(f_ref, xc_ref, yc_ref, w1f_ref, w1x_ref, w1y_ref, b1_ref, a_ref):
    f = f_ref[0]
    xc = xc_ref[0]
    yc = yc_ref[0]
    a = jnp.dot(f, w1f_ref[...], preferred_element_type=jnp.float32)
    a = a + xc * w1x_ref[...] + yc * w1y_ref[...] + b1_ref[...]
    a_ref[0] = jnp.pad(a, ((0, 0), (0, TW - HIDDEN)))


def _precompute(feats, xcol, ycol, w1f, w1x, w1y, b1r):
    return pl.pallas_call(
        _pre_body,
        grid=(B,),
        in_specs=[
            pl.BlockSpec((1, N, IN_CH), lambda b: (b, 0, 0)),
            pl.BlockSpec((1, N, 1), lambda b: (b, 0, 0)),
            pl.BlockSpec((1, N, 1), lambda b: (b, 0, 0)),
            pl.BlockSpec((IN_CH, HIDDEN), lambda b: (0, 0)),
            pl.BlockSpec((1, HIDDEN), lambda b: (0, 0)),
            pl.BlockSpec((1, HIDDEN), lambda b: (0, 0)),
            pl.BlockSpec((1, HIDDEN), lambda b: (0, 0)),
        ],
        out_specs=pl.BlockSpec((1, N, TW), lambda b: (b, 0, 0)),
        out_shape=jax.ShapeDtypeStruct((B, N, TW), jnp.float32),
    )(feats, xcol, ycol, w1f, w1x, w1y, b1r)


def _sc_gather_body(table_hbm, idx_hbm, out_hbm, idx_v, rows_v, sem):
    wid = lax.axis_index("s") * 2 + lax.axis_index("c")
    pltpu.sync_copy(idx_hbm.at[wid], idx_v)

    def chunk(j, _):
        pltpu.async_copy(table_hbm.at[idx_v.at[j]], rows_v, sem).wait()
        pltpu.sync_copy(rows_v,
                        out_hbm.at[pl.ds(wid * ROWS_W + j * CHUNK, CHUNK)])
        return 0

    jax.lax.fori_loop(0, NCHUNK, chunk, 0)


_sc_gather = functools.partial(
    pl.kernel,
    mesh=plsc.VectorSubcoreMesh(core_axis_name="c", subcore_axis_name="s"),
    out_type=jax.ShapeDtypeStruct((TOT, TW), jnp.float32),
    scratch_types=[
        pltpu.VMEM((NCHUNK, CHUNK), jnp.int32),
        pltpu.VMEM((CHUNK, TW), jnp.float32),
        pltpu.SemaphoreType.DMA,
    ],
)(_sc_gather_body)


def _mlp_body(ag_ref, val_ref, cx_ref, cy_ref, w1x_ref, w1y_ref,
              g1_ref, be1_ref, w2_ref, b2_ref, out_ref):
    ag = ag_ref[0, 0, :, :HIDDEN]
    val = val_ref[0, 0]
    cx = cx_ref[0, 0]
    cy = cy_ref[0, 0]
    cs = cx * w1x_ref[...] + cy * w1y_ref[...]
    cs_rep = jnp.broadcast_to(cs[:, None, :],
                              (T, K, HIDDEN)).reshape(T * K, HIDDEN)
    h = ag - cs_rep
    mu = jnp.mean(h, axis=1, keepdims=True)
    hm = h - mu
    var = jnp.mean(hm * hm, axis=1, keepdims=True)
    h = hm / jnp.sqrt(var + 1e-5) * g1_ref[...] + be1_ref[...]
    h = jnp.maximum(h, 0.0)
    o = jnp.dot(h, w2_ref[...], preferred_element_type=jnp.float32)
    o = o + b2_ref[...]
    vmask = jnp.broadcast_to(val[:, :, None] != 0, (T, K, OUT_CH))
    o = jnp.where(vmask, o.reshape(T, K, OUT_CH), -10000.0)
    out_ref[0, 0] = jnp.max(o, axis=1)


def _mlp(ag4, val, cxc, cyc, w1x, w1y, g1r, be1r, W2, b2r):
    grid = (B, S // T)
    return pl.pallas_call(
        _mlp_body,
        grid=grid,
        in_specs=[
            pl.BlockSpec((1, 1, T * K, TW), lambda b, t: (b, t, 0, 0)),
            pl.BlockSpec((1, 1, T, K), lambda b, t: (b, t, 0, 0)),
            pl.BlockSpec((1, 1, T, 1), lambda b, t: (b, t, 0, 0)),
            pl.BlockSpec((1, 1, T, 1), lambda b, t: (b, t, 0, 0)),
            pl.BlockSpec((1, HIDDEN), lambda b, t: (0, 0)),
            pl.BlockSpec((1, HIDDEN), lambda b, t: (0, 0)),
            pl.BlockSpec((1, HIDDEN), lambda b, t: (0, 0)),
            pl.BlockSpec((1, HIDDEN), lambda b, t: (0, 0)),
            pl.BlockSpec((HIDDEN, OUT_CH), lambda b, t: (0, 0)),
            pl.BlockSpec((1, OUT_CH), lambda b, t: (0, 0)),
        ],
        out_specs=pl.BlockSpec((1, 1, T, OUT_CH), lambda b, t: (b, t, 0, 0)),
        out_shape=jax.ShapeDtypeStruct((B, S // T, T, OUT_CH), jnp.float32),
    )(ag4, val, cxc, cyc, w1x, w1y, g1r, be1r, W2, b2r)


@jax.jit
def kernel(xyz, feats, W1, b1, g1, be1, W2, b2):
    x = xyz[:, :, 0]
    y = xyz[:, :, 1]
    xy = jnp.concatenate([x, y], axis=0)
    x3 = x.reshape(B, 1, N)
    y3 = y.reshape(B, 1, N)
    xcol = x.reshape(B, N, 1)
    ycol = y.reshape(B, N, 1)
    w1x = W1[0].reshape(1, HIDDEN)
    w1y = W1[1].reshape(1, HIDDEN)
    w1f = W1[2:]
    b1r = b1.reshape(1, HIDDEN)
    g1r = g1.reshape(1, HIDDEN)
    be1r = be1.reshape(1, HIDDEN)
    b2r = b2.reshape(1, OUT_CH)

    cx, cy = _fps(xy)
    cxc = cx.reshape(B, S // T, T, 1)
    cyc = cy.reshape(B, S // T, T, 1)
    gidx, valmask = _ball_query(x3, y3, cxc, cyc)
    a = _precompute(feats, xcol, ycol, w1f, w1x, w1y, b1r)
    ag = _sc_gather(a.reshape(B * N, TW),
                    gidx.reshape(NW, NCHUNK, CHUNK))
    ag4 = ag.reshape(B, S // T, T * K, TW)
    out = _mlp(ag4, valmask, cxc, cyc, w1x, w1y, g1r, be1r, W2, b2r)

    centers = jnp.stack([cx, cy], axis=-1)
    return centers, out.reshape(B, S, OUT_CH)
